# Initial kernel scaffold; baseline (speedup 1.0000x reference)
import jax
import jax.numpy as jnp
from jax.experimental import pallas as pl


def kernel(x, router_w1, router_b1, router_w2, router_b2, gamma, beta, penalty):
    raise NotImplementedError("write your pallas kernel here")


# speedup vs baseline: 1.2156x; 1.2156x over previous
import functools

import jax
import jax.numpy as jnp
from jax.experimental import pallas as pl
from jax.experimental.pallas import tpu as pltpu

B, S, D = 4, 2048, 4096
E, K = 8, 2

S_CHUNK = 256
D_CHUNK = 512


def _pool_kernel(x_ref, out_ref):
    s = pl.program_id(1)

    @pl.when(s == 0)
    def _():
        out_ref[...] = jnp.zeros_like(out_ref)

    out_ref[...] += jnp.sum(x_ref[...], axis=1, keepdims=True) * (1.0 / S)


def _router_kernel(pooled_ref, w1_ref, b1_ref, w2_ref, b2_ref, pen_ref,
                   gamma_ref, beta_ref,
                   coeff_ref, lb_ref, g_ref, b_ref, h_scr):
    j = pl.program_id(0)
    nj = pl.num_programs(0)
    part = jnp.dot(pooled_ref[...], w1_ref[...],
                   preferred_element_type=jnp.float32) + b1_ref[...][None, :]
    h_scr[:, pl.ds(j * D_CHUNK, D_CHUNK)] = jnp.maximum(part, 0.0)

    @pl.when(j == nj - 1)
    def _():
        logits = jnp.dot(h_scr[...], w2_ref[...],
                         preferred_element_type=jnp.float32) + b2_ref[...][None, :]
        m = jnp.max(logits, axis=-1, keepdims=True)
        ex = jnp.exp(logits - m)
        route_prob = ex / jnp.sum(ex, axis=-1, keepdims=True)
        biased = route_prob - pen_ref[...][None, :]

        eidx = jax.lax.broadcasted_iota(jnp.int32, (B, E), 1)
        big = jnp.int32(E)
        m1 = jnp.max(biased, axis=-1, keepdims=True)
        i1 = jnp.min(jnp.where(biased == m1, eidx, big), axis=-1, keepdims=True)
        masked = jnp.where(eidx == i1, -jnp.inf, biased)
        m2 = jnp.max(masked, axis=-1, keepdims=True)
        i2 = jnp.min(jnp.where(masked == m2, eidx, big), axis=-1, keepdims=True)

        denom = m1 + m2
        w1v = m1 / denom
        w2v = m2 / denom
        is1 = (eidx == i1)
        is2 = (eidx == i2)
        coeff = jnp.where(is1, w1v, 0.0) + jnp.where(is2, w2v, 0.0)
        coeff_ref[...] = coeff

        cnt = jnp.sum(is1.astype(jnp.float32) + is2.astype(jnp.float32),
                      axis=0, keepdims=True)
        importance = jnp.mean(route_prob, axis=0, keepdims=True)
        load = cnt / jnp.maximum(jnp.sum(cnt), 1.0)
        lb_ref[...] = jnp.float32(E) * jnp.sum(importance * load, keepdims=True
                                               ).reshape(1, 1)

        g_ref[...] = jnp.dot(coeff, gamma_ref[...],
                             preferred_element_type=jnp.float32)
        b_ref[...] = jnp.dot(coeff, beta_ref[...],
                             preferred_element_type=jnp.float32)


def _norm_kernel(x_ref, g_ref, b_ref, o_ref):
    xb = x_ref[0]
    mu = jnp.mean(xb, axis=-1, keepdims=True)
    xc = xb - mu
    var = jnp.mean(xc * xc, axis=-1, keepdims=True)
    xn = xc * jax.lax.rsqrt(var + 1e-6)
    o_ref[0] = xn * g_ref[0] + b_ref[0]


@jax.jit
def kernel(x, router_w1, router_b1, router_w2, router_b2, gamma, beta, penalty):
    pooled3 = pl.pallas_call(
        _pool_kernel,
        grid=(B, S // S_CHUNK),
        in_specs=[pl.BlockSpec((1, S_CHUNK, D), lambda b, s: (b, s, 0))],
        out_specs=pl.BlockSpec((1, 1, D), lambda b, s: (b, 0, 0)),
        out_shape=jax.ShapeDtypeStruct((B, 1, D), jnp.float32),
        compiler_params=pltpu.CompilerParams(dimension_semantics=("parallel", "arbitrary")),
    )(x)
    pooled = pooled3.reshape(B, D)

    coeff, lb, g, b = pl.pallas_call(
        _router_kernel,
        grid=(D // D_CHUNK,),
        in_specs=[
            pl.BlockSpec((B, D), lambda j: (0, 0)),
            pl.BlockSpec((D, D_CHUNK), lambda j: (0, j)),
            pl.BlockSpec((D_CHUNK,), lambda j: (j,)),
            pl.BlockSpec((D, E), lambda j: (0, 0)),
            pl.BlockSpec((E,), lambda j: (0,)),
            pl.BlockSpec((E,), lambda j: (0,)),
            pl.BlockSpec((E, D), lambda j: (0, 0)),
            pl.BlockSpec((E, D), lambda j: (0, 0)),
        ],
        out_specs=[
            pl.BlockSpec((B, E), lambda j: (0, 0)),
            pl.BlockSpec((1, 1), lambda j: (0, 0)),
            pl.BlockSpec((B, D), lambda j: (0, 0)),
            pl.BlockSpec((B, D), lambda j: (0, 0)),
        ],
        out_shape=[
            jax.ShapeDtypeStruct((B, E), jnp.float32),
            jax.ShapeDtypeStruct((1, 1), jnp.float32),
            jax.ShapeDtypeStruct((B, D), jnp.float32),
            jax.ShapeDtypeStruct((B, D), jnp.float32),
        ],
        scratch_shapes=[pltpu.VMEM((B, D), jnp.float32)],
        compiler_params=pltpu.CompilerParams(dimension_semantics=("arbitrary",)),
    )(pooled, router_w1, router_b1, router_w2, router_b2, penalty, gamma, beta)

    out = pl.pallas_call(
        _norm_kernel,
        grid=(B, S // S_CHUNK),
        in_specs=[
            pl.BlockSpec((1, S_CHUNK, D), lambda b, s: (b, s, 0)),
            pl.BlockSpec((1, 1, D), lambda b, s: (b, 0, 0)),
            pl.BlockSpec((1, 1, D), lambda b, s: (b, 0, 0)),
        ],
        out_specs=pl.BlockSpec((1, S_CHUNK, D), lambda b, s: (b, s, 0)),
        out_shape=jax.ShapeDtypeStruct((B, S, D), jnp.float32),
        compiler_params=pltpu.CompilerParams(dimension_semantics=("parallel", "arbitrary")),
    )(x, g.reshape(B, 1, D), b.reshape(B, 1, D))

    return (out, coeff, lb.reshape(()))


# speedup vs baseline: 1.2412x; 1.0211x over previous
import jax
import jax.numpy as jnp
from jax.experimental import pallas as pl
from jax.experimental.pallas import tpu as pltpu

B, S, D = 4, 2048, 4096
E, K = 8, 2

S_CHUNK = 512
D_CHUNK = 512


def _pool_kernel(x_ref, out_ref):
    s = pl.program_id(1)

    @pl.when(s == 0)
    def _():
        out_ref[...] = jnp.zeros_like(out_ref)

    out_ref[...] += jnp.sum(x_ref[...], axis=1, keepdims=True) * (1.0 / S)


def _matmul_kernel(pooled_ref, w1_ref, b1_ref, h_ref, acc):
    j = pl.program_id(0)
    nj = pl.num_programs(0)

    @pl.when(j == 0)
    def _():
        acc[...] = jnp.zeros_like(acc)

    acc[...] += jnp.dot(pooled_ref[:, pl.ds(j * D_CHUNK, D_CHUNK)], w1_ref[...],
                        preferred_element_type=jnp.float32)

    @pl.when(j == nj - 1)
    def _():
        h_ref[...] = jnp.maximum(acc[...] + b1_ref[...][None, :], 0.0)


def _routing_tail(h, w2, b2, pen, gamma, beta):
    logits = jnp.dot(h, w2, preferred_element_type=jnp.float32) + b2[None, :]
    m = jnp.max(logits, axis=-1, keepdims=True)
    ex = jnp.exp(logits - m)
    route_prob = ex / jnp.sum(ex, axis=-1, keepdims=True)
    biased = route_prob - pen[None, :]

    eidx = jax.lax.broadcasted_iota(jnp.int32, (B, E), 1)
    big = jnp.int32(E)
    m1 = jnp.max(biased, axis=-1, keepdims=True)
    i1 = jnp.min(jnp.where(biased == m1, eidx, big), axis=-1, keepdims=True)
    masked = jnp.where(eidx == i1, -jnp.inf, biased)
    m2 = jnp.max(masked, axis=-1, keepdims=True)
    i2 = jnp.min(jnp.where(masked == m2, eidx, big), axis=-1, keepdims=True)

    denom = m1 + m2
    is1 = (eidx == i1)
    is2 = (eidx == i2)
    coeff = jnp.where(is1, m1 / denom, 0.0) + jnp.where(is2, m2 / denom, 0.0)

    cnt = jnp.sum(is1.astype(jnp.float32) + is2.astype(jnp.float32),
                  axis=0, keepdims=True)
    importance = jnp.mean(route_prob, axis=0, keepdims=True)
    load = cnt / jnp.maximum(jnp.sum(cnt), 1.0)
    lb = jnp.float32(E) * jnp.sum(importance * load)

    g = jnp.dot(coeff, gamma, preferred_element_type=jnp.float32)
    bvec = jnp.dot(coeff, beta, preferred_element_type=jnp.float32)
    return coeff, lb, g, bvec


def _norm_kernel(h_ref, w2_ref, b2_ref, pen_ref, gamma_ref, beta_ref, x_ref,
                 o_ref, coeff_ref, lb_ref, gb_scr):
    b = pl.program_id(0)
    s = pl.program_id(1)

    @pl.when((b == 0) & (s == 0))
    def _():
        coeff, lb, g, bvec = _routing_tail(
            h_ref[...], w2_ref[...], b2_ref[...], pen_ref[...],
            gamma_ref[...], beta_ref[...])
        coeff_ref[...] = coeff
        lb_ref[...] = lb.reshape(1, 1)
        gb_scr[0] = g
        gb_scr[1] = bvec

    xb = x_ref[0]
    mu = jnp.mean(xb, axis=-1, keepdims=True)
    xc = xb - mu
    var = jnp.mean(xc * xc, axis=-1, keepdims=True)
    xn = xc * jax.lax.rsqrt(var + 1e-6)
    o_ref[0] = xn * gb_scr[0, b][None, :] + gb_scr[1, b][None, :]


@jax.jit
def kernel(x, router_w1, router_b1, router_w2, router_b2, gamma, beta, penalty):
    pooled3 = pl.pallas_call(
        _pool_kernel,
        grid=(B, S // S_CHUNK),
        in_specs=[pl.BlockSpec((1, S_CHUNK, D), lambda b, s: (b, s, 0))],
        out_specs=pl.BlockSpec((1, 1, D), lambda b, s: (b, 0, 0)),
        out_shape=jax.ShapeDtypeStruct((B, 1, D), jnp.float32),
        compiler_params=pltpu.CompilerParams(dimension_semantics=("parallel", "arbitrary")),
    )(x)
    pooled = pooled3.reshape(B, D)

    h = pl.pallas_call(
        _matmul_kernel,
        grid=(D // D_CHUNK,),
        in_specs=[
            pl.BlockSpec((B, D), lambda j: (0, 0)),
            pl.BlockSpec((D_CHUNK, D), lambda j: (j, 0)),
            pl.BlockSpec((D,), lambda j: (0,)),
        ],
        out_specs=pl.BlockSpec((B, D), lambda j: (0, 0)),
        out_shape=jax.ShapeDtypeStruct((B, D), jnp.float32),
        scratch_shapes=[pltpu.VMEM((B, D), jnp.float32)],
        compiler_params=pltpu.CompilerParams(dimension_semantics=("arbitrary",)),
    )(pooled, router_w1, router_b1)

    out, coeff, lb = pl.pallas_call(
        _norm_kernel,
        grid=(B, S // S_CHUNK),
        in_specs=[
            pl.BlockSpec((B, D), lambda b, s: (0, 0)),
            pl.BlockSpec((D, E), lambda b, s: (0, 0)),
            pl.BlockSpec((E,), lambda b, s: (0,)),
            pl.BlockSpec((E,), lambda b, s: (0,)),
            pl.BlockSpec((E, D), lambda b, s: (0, 0)),
            pl.BlockSpec((E, D), lambda b, s: (0, 0)),
            pl.BlockSpec((1, S_CHUNK, D), lambda b, s: (b, s, 0)),
        ],
        out_specs=[
            pl.BlockSpec((1, S_CHUNK, D), lambda b, s: (b, s, 0)),
            pl.BlockSpec((B, E), lambda b, s: (0, 0)),
            pl.BlockSpec((1, 1), lambda b, s: (0, 0)),
        ],
        out_shape=[
            jax.ShapeDtypeStruct((B, S, D), jnp.float32),
            jax.ShapeDtypeStruct((B, E), jnp.float32),
            jax.ShapeDtypeStruct((1, 1), jnp.float32),
        ],
        scratch_shapes=[pltpu.VMEM((2, B, D), jnp.float32)],
        compiler_params=pltpu.CompilerParams(dimension_semantics=("arbitrary", "arbitrary")),
    )(h, router_w2, router_b2, penalty, gamma, beta, x)

    return (out, coeff, lb.reshape(()))
